# Initial kernel scaffold; baseline (speedup 1.0000x reference)
import jax
import jax.numpy as jnp
from jax.experimental import pallas as pl


def kernel(in_feat, edge_index, W1, b1, W2, b2, W3, b3, Wc, bc):
    raise NotImplementedError("write your pallas kernel here")


# speedup vs baseline: 4.2505x; 4.2505x over previous
import functools

import jax
import jax.numpy as jnp
from jax import lax
from jax.experimental import pallas as pl
from jax.experimental.pallas import tpu as pltpu
from jax.experimental.pallas import tpu_sc as plsc

N = 10000
E = 160000
F = 256
H = 512
NP = 10240
HALF = NP // 2
ACC_ROWS = HALF + 128
NCLS = 10

_mesh = plsc.VectorSubcoreMesh(core_axis_name="c", subcore_axis_name="s")


def _iota16():
    return lax.iota(jnp.int32, 16)


@functools.partial(
    pl.kernel, mesh=_mesh,
    out_type=[jax.ShapeDtypeStruct((2, NP), jnp.float32),
              jax.ShapeDtypeStruct((2, NP), jnp.float32)],
    scratch_types=[
        pltpu.VMEM((5008,), jnp.int32),
        pltpu.VMEM((5008,), jnp.int32),
        pltpu.VMEM((128,), jnp.int32),
        pltpu.VMEM((128,), jnp.int32),
        pltpu.VMEM((128,), jnp.float32),
        pltpu.VMEM((16,), jnp.int32),
        pltpu.VMEM((16,), jnp.int32),
        pltpu.VMEM((16,), jnp.float32),
        pltpu.VMEM_SHARED((NP,), jnp.float32),
        pltpu.VMEM_SHARED((NP,), jnp.float32),
    ],
)
def _deg_kernel(esrc, edst, zrow, do_out, di_out,
                sbuf, dbuf, iob, iib, onesb, tio, tii, tv, acco, acci):
    cid = lax.axis_index("c")
    sid = lax.axis_index("s")
    tid = cid * 16 + sid
    pltpu.sync_copy(zrow, acco.at[pl.ds(sid * 640, 640)])
    pltpu.sync_copy(zrow, acci.at[pl.ds(sid * 640, 640)])
    onev = jnp.ones((16,), jnp.float32)

    def fill_ones(i, _):
        onesb[pl.ds(i * 16, 16)] = onev
        return 0
    lax.fori_loop(0, 8, fill_ones, 0)
    z16 = jnp.zeros((16,), jnp.int32)
    sbuf[pl.ds(4992, 16)] = z16
    dbuf[pl.ds(4992, 16)] = z16
    base = tid * 5000
    pltpu.sync_copy(esrc.at[pl.ds(base, 5000)], sbuf.at[pl.ds(0, 5000)])
    pltpu.sync_copy(edst.at[pl.ds(base, 5000)], dbuf.at[pl.ds(0, 5000)])
    plsc.subcore_barrier()

    def batch(b, _):
        for j in range(8):
            off = b * 128 + 16 * j
            s16 = sbuf[pl.ds(off, 16)]
            d16 = dbuf[pl.ds(off, 16)]
            iob[pl.ds(16 * j, 16)] = jnp.clip(s16, 0, NP - 1)
            iib[pl.ds(16 * j, 16)] = jnp.clip(d16, 0, NP - 1)
        pltpu.sync_copy(onesb, acco.at[iob], add=True)
        pltpu.sync_copy(onesb, acci.at[iib], add=True)
        return 0
    lax.fori_loop(0, 39, batch, 0)
    valid = _iota16() < 8
    s16 = sbuf[pl.ds(4992, 16)]
    d16 = dbuf[pl.ds(4992, 16)]
    tio[...] = jnp.clip(s16, 0, NP - 1)
    tii[...] = jnp.clip(d16, 0, NP - 1)
    tv[...] = jnp.where(valid, 1.0, 0.0).astype(jnp.float32)
    pltpu.sync_copy(tv, acco.at[tio], add=True)
    pltpu.sync_copy(tv, acci.at[tii], add=True)
    plsc.subcore_barrier()
    pltpu.sync_copy(acco.at[pl.ds(sid * 640, 640)],
                    do_out.at[cid, pl.ds(sid * 640, 640)])
    pltpu.sync_copy(acci.at[pl.ds(sid * 640, 640)],
                    di_out.at[cid, pl.ds(sid * 640, 640)])


NPG = NP + 128


@functools.partial(
    pl.kernel, mesh=_mesh,
    out_type=[jax.ShapeDtypeStruct((2, NPG), jnp.float32)],
    scratch_types=[
        pltpu.VMEM((5008,), jnp.int32),
        pltpu.VMEM((5008,), jnp.int32),
        pltpu.VMEM((128,), jnp.int32),
        pltpu.VMEM((128,), jnp.int32),
        pltpu.VMEM((128,), jnp.float32),
        pltpu.VMEM_SHARED((NPG,), jnp.float32),
        pltpu.SemaphoreType.DMA,
    ],
)
def _cvec_kernel(esrc, edst, ni, zrow, c_out, sbuf, dbuf, ib, db, vb, acc, sem):
    cid = lax.axis_index("c")
    sid = lax.axis_index("s")
    tid = cid * 16 + sid
    pltpu.sync_copy(zrow, acc.at[pl.ds(sid * 640, 640)])

    @pl.when(sid == 0)
    def _():
        pltpu.sync_copy(zrow.at[pl.ds(0, 128)], acc.at[pl.ds(NP, 128)])
    z16 = jnp.zeros((16,), jnp.int32)
    sbuf[pl.ds(4992, 16)] = z16
    dbuf[pl.ds(4992, 16)] = z16
    base = tid * 5000
    pltpu.sync_copy(esrc.at[pl.ds(base, 5000)], sbuf.at[pl.ds(0, 5000)])
    pltpu.sync_copy(edst.at[pl.ds(base, 5000)], dbuf.at[pl.ds(0, 5000)])
    plsc.subcore_barrier()
    iot = _iota16()
    garb = NP + sid * 4 + (iot & 3)

    def batch(b, _):
        for j in range(8):
            off = b * 128 + 16 * j
            s16 = sbuf[pl.ds(off, 16)]
            d16 = dbuf[pl.ds(off, 16)]
            ib[pl.ds(16 * j, 16)] = jnp.clip(s16, 0, NP - 1)
            db[pl.ds(16 * j, 16)] = jnp.clip(d16, 0, NP - 1)
        pltpu.async_copy(ni.at[db], vb, sem).wait()
        pltpu.sync_copy(vb, acc.at[ib], add=True)
        return 0
    lax.fori_loop(0, 39, batch, 0)
    valid = iot < 8
    s16 = jnp.clip(sbuf[pl.ds(4992, 16)], 0, NP - 1)
    d16 = jnp.clip(dbuf[pl.ds(4992, 16)], 0, NP - 1)
    ib[pl.ds(0, 16)] = jnp.where(valid, s16, garb)
    db[pl.ds(0, 16)] = d16

    def padrest(i, _):
        ib[pl.ds(16 + i * 16, 16)] = garb
        db[pl.ds(16 + i * 16, 16)] = z16
        return 0
    lax.fori_loop(0, 7, padrest, 0)
    pltpu.async_copy(ni.at[db], vb, sem).wait()
    pltpu.sync_copy(vb, acc.at[ib], add=True)
    plsc.subcore_barrier()
    pltpu.sync_copy(acc.at[pl.ds(sid * 640, 640)],
                    c_out.at[cid, pl.ds(sid * 640, 640)])

    @pl.when(sid == 0)
    def _():
        pltpu.sync_copy(acc.at[pl.ds(NP, 128)], c_out.at[cid, pl.ds(NP, 128)])


@functools.partial(
    pl.kernel, mesh=_mesh,
    out_type=[jax.ShapeDtypeStruct((NP, 128), jnp.float32),
              jax.ShapeDtypeStruct((NP, 128), jnp.float32)],
    scratch_types=[
        pltpu.VMEM((2000,), jnp.int32),
        pltpu.VMEM((2000,), jnp.int32),
        pltpu.VMEM((80,), jnp.int32),
        pltpu.VMEM((80,), jnp.int32),
        pltpu.VMEM((80, 128), jnp.float32),
        pltpu.VMEM((80, 128), jnp.float32),
        pltpu.VMEM_SHARED((ACC_ROWS, 128), jnp.float32),
        pltpu.VMEM_SHARED((ACC_ROWS, 128), jnp.float32),
        pltpu.SemaphoreType.DMA,
        pltpu.SemaphoreType.DMA,
    ],
)
def _prop_kernel(xa, xb, esrc, edst, zstripe, ya, yb,
                 sbuf, dbuf, gf, df, stg_a, stg_b, acc_a, acc_b, sema, semb):
    cid = lax.axis_index("c")
    sid = lax.axis_index("s")
    lo = cid * HALF
    pltpu.sync_copy(zstripe, acc_a.at[pl.ds(sid * 328, 328)])
    pltpu.sync_copy(zstripe, acc_b.at[pl.ds(sid * 328, 328)])
    plsc.subcore_barrier()
    iot = _iota16()
    garb = HALF + sid * 8 + (iot & 7)

    def chunk(c, _):
        ebase = sid * 10000 + c * 2000
        pltpu.sync_copy(esrc.at[pl.ds(ebase, 2000)], sbuf)
        pltpu.sync_copy(edst.at[pl.ds(ebase, 2000)], dbuf)

        def batch(b, _2):
            for j in range(5):
                off = b * 80 + 16 * j
                s16 = jnp.clip(sbuf[pl.ds(off, 16)], 0, NP - 1)
                d16 = dbuf[pl.ds(off, 16)]
                m = (d16 >= lo) & (d16 < lo + HALF)
                gf[pl.ds(16 * j, 16)] = s16
                df[pl.ds(16 * j, 16)] = jnp.where(m, d16 - lo, garb)
            cpa = pltpu.async_copy(xa.at[gf], stg_a, sema)
            cpb = pltpu.async_copy(xb.at[gf], stg_b, semb)
            cpa.wait()
            pltpu.sync_copy(stg_a, acc_a.at[df], add=True)
            cpb.wait()
            pltpu.sync_copy(stg_b, acc_b.at[df], add=True)
            return 0
        lax.fori_loop(0, 25, batch, 0)
        return 0
    lax.fori_loop(0, 5, chunk, 0)
    plsc.subcore_barrier()
    row0 = sid * 320
    for k in range(4):
        pltpu.sync_copy(acc_a.at[pl.ds(row0 + 80 * k, 80)],
                        ya.at[pl.ds(cid * HALF + row0 + 80 * k, 80)])
        pltpu.sync_copy(acc_b.at[pl.ds(row0 + 80 * k, 80)],
                        yb.at[pl.ds(cid * HALF + row0 + 80 * k, 80)])


def _scale_body(xf, dop, dip, x0a, x0b, no_o, ni_o):
    do = dop[0] + dop[1]
    di = dip[0] + dip[1]
    no = lax.rsqrt(jnp.clip(do, 1.0, None))
    ni = lax.rsqrt(jnp.clip(di, 1.0, None))
    no_o[...] = no
    ni_o[...] = ni
    x0 = xf[...] * no
    x0a[...] = x0[:, :128]
    x0b[...] = x0[:, 128:]


def _scale(xpad, do3, di3):
    return pl.pallas_call(
        _scale_body,
        grid=(16,),
        in_specs=[
            pl.BlockSpec((640, F), lambda b: (b, 0)),
            pl.BlockSpec((2, 640, 1), lambda b: (0, b, 0)),
            pl.BlockSpec((2, 640, 1), lambda b: (0, b, 0)),
        ],
        out_specs=[
            pl.BlockSpec((640, 128), lambda b: (b, 0)),
            pl.BlockSpec((640, 128), lambda b: (b, 0)),
            pl.BlockSpec((640, 1), lambda b: (b, 0)),
            pl.BlockSpec((640, 1), lambda b: (b, 0)),
        ],
        out_shape=[
            jax.ShapeDtypeStruct((NP, 128), jnp.float32),
            jax.ShapeDtypeStruct((NP, 128), jnp.float32),
            jax.ShapeDtypeStruct((NP, 1), jnp.float32),
            jax.ShapeDtypeStruct((NP, 1), jnp.float32),
        ],
    )(xpad, do3, di3)


def _mlp1_body(p1a, p1b, ni, no, w1, b1, xa, xb, xc, xd):
    nic = ni[...]
    noc = no[...]
    w1full = w1[...]
    h = jnp.dot(p1a[...] * nic, w1full[:128], preferred_element_type=jnp.float32)
    h += jnp.dot(p1b[...] * nic, w1full[128:], preferred_element_type=jnp.float32)
    h = jnp.maximum(h + b1[...], 0.0) * noc
    xa[...] = h[:, :128]
    xb[...] = h[:, 128:256]
    xc[...] = h[:, 256:384]
    xd[...] = h[:, 384:]


def _mlp1(p1a, p1b, ni2, no2, W1, b1):
    return pl.pallas_call(
        _mlp1_body,
        grid=(20,),
        in_specs=[
            pl.BlockSpec((512, 128), lambda b: (b, 0)),
            pl.BlockSpec((512, 128), lambda b: (b, 0)),
            pl.BlockSpec((512, 1), lambda b: (b, 0)),
            pl.BlockSpec((512, 1), lambda b: (b, 0)),
            pl.BlockSpec((F, H), lambda b: (0, 0)),
            pl.BlockSpec((1, H), lambda b: (0, 0)),
        ],
        out_specs=[
            pl.BlockSpec((512, 128), lambda b: (b, 0)),
            pl.BlockSpec((512, 128), lambda b: (b, 0)),
            pl.BlockSpec((512, 128), lambda b: (b, 0)),
            pl.BlockSpec((512, 128), lambda b: (b, 0)),
        ],
        out_shape=[
            jax.ShapeDtypeStruct((NP, 128), jnp.float32),
            jax.ShapeDtypeStruct((NP, 128), jnp.float32),
            jax.ShapeDtypeStruct((NP, 128), jnp.float32),
            jax.ShapeDtypeStruct((NP, 128), jnp.float32),
        ],
    )(p1a, p1b, ni2, no2, W1, b1)


def _mlp2_body(p2a, p2b, p2c, p2d, ni, no, cp, w2, b2, w3, wc, b3, bc, out, accv):
    b = pl.program_id(0)
    nic = ni[...]
    w2full = w2[...]
    h = jnp.dot(p2a[...] * nic, w2full[:128], preferred_element_type=jnp.float32)
    h += jnp.dot(p2b[...] * nic, w2full[128:256], preferred_element_type=jnp.float32)
    h += jnp.dot(p2c[...] * nic, w2full[256:384], preferred_element_type=jnp.float32)
    h += jnp.dot(p2d[...] * nic, w2full[384:], preferred_element_type=jnp.float32)
    h = jnp.maximum(h + b2[...], 0.0)
    c = cp[0] + cp[1]
    w = (c * no[...]) * (1.0 / N)
    v = jnp.sum(h * w, axis=0, keepdims=True)

    @pl.when(b == 0)
    def _():
        accv[...] = jnp.zeros_like(accv)
    accv[...] += v

    @pl.when(b == 19)
    def _():
        hg = jnp.dot(accv[...], w3[...], preferred_element_type=jnp.float32) + b3[...]
        out[...] = jnp.dot(hg, wc[...], preferred_element_type=jnp.float32) + bc[...]


def _mlp2(p2a, p2b, p2c, p2d, ni2, no2, cp3, W2, b2, W3, Wc, b3, bc):
    return pl.pallas_call(
        _mlp2_body,
        grid=(20,),
        in_specs=[
            pl.BlockSpec((512, 128), lambda b: (b, 0)),
            pl.BlockSpec((512, 128), lambda b: (b, 0)),
            pl.BlockSpec((512, 128), lambda b: (b, 0)),
            pl.BlockSpec((512, 128), lambda b: (b, 0)),
            pl.BlockSpec((512, 1), lambda b: (b, 0)),
            pl.BlockSpec((512, 1), lambda b: (b, 0)),
            pl.BlockSpec((2, 512, 1), lambda b: (0, b, 0)),
            pl.BlockSpec((H, H), lambda b: (0, 0)),
            pl.BlockSpec((1, H), lambda b: (0, 0)),
            pl.BlockSpec((H, H), lambda b: (0, 0)),
            pl.BlockSpec((H, NCLS), lambda b: (0, 0)),
            pl.BlockSpec((1, H), lambda b: (0, 0)),
            pl.BlockSpec((1, NCLS), lambda b: (0, 0)),
        ],
        out_specs=pl.BlockSpec((1, NCLS), lambda b: (0, 0)),
        out_shape=jax.ShapeDtypeStruct((1, NCLS), jnp.float32),
        scratch_shapes=[pltpu.VMEM((1, H), jnp.float32)],
    )(p2a, p2b, p2c, p2d, ni2, no2, cp3, W2, b2, W3, Wc, b3, bc)


def kernel(in_feat, edge_index, W1, b1, W2, b2, W3, b3, Wc, bc):
    f32 = jnp.float32
    xpad = jnp.pad(in_feat, ((0, NP - N), (0, 0)))
    zrow = jnp.zeros((640,), f32)
    zstripe = jnp.zeros((328, 128), f32)
    esrc = edge_index[0]
    edst = edge_index[1]

    do_p, di_p = _deg_kernel(esrc, edst, zrow)
    do3 = do_p.reshape(2, NP, 1)
    di3 = di_p.reshape(2, NP, 1)
    x0a, x0b, no2, ni2 = _scale(xpad, do3, di3)
    (c_pg,) = _cvec_kernel(esrc, edst, ni2.reshape(NP), zrow)
    c_p = c_pg[:, :NP]
    p1a, p1b = _prop_kernel(x0a, x0b, esrc, edst, zstripe)
    x1a, x1b, x1c, x1d = _mlp1(p1a, p1b, ni2, no2, W1, b1.reshape(1, H))
    p2a, p2b = _prop_kernel(x1a, x1b, esrc, edst, zstripe)
    p2c, p2d = _prop_kernel(x1c, x1d, esrc, edst, zstripe)
    out = _mlp2(p2a, p2b, p2c, p2d, ni2, no2, c_p.reshape(2, NP, 1),
                W2, b2.reshape(1, H), W3, Wc, b3.reshape(1, H),
                bc.reshape(1, NCLS))
    return out


# speedup vs baseline: 4.4163x; 1.0390x over previous
import functools

import jax
import jax.numpy as jnp
from jax import lax
from jax.experimental import pallas as pl
from jax.experimental.pallas import tpu as pltpu
from jax.experimental.pallas import tpu_sc as plsc

N = 10000
E = 160000
F = 256
H = 512
NP = 10240
HALF = NP // 2
ACC_ROWS = HALF + 128
NCLS = 10

_mesh = plsc.VectorSubcoreMesh(core_axis_name="c", subcore_axis_name="s")


def _iota16():
    return lax.iota(jnp.int32, 16)


@functools.partial(
    pl.kernel, mesh=_mesh,
    out_type=[jax.ShapeDtypeStruct((2, NP), jnp.float32),
              jax.ShapeDtypeStruct((2, NP), jnp.float32)],
    scratch_types=[
        pltpu.VMEM((5008,), jnp.int32),
        pltpu.VMEM((5008,), jnp.int32),
        pltpu.VMEM((128,), jnp.int32),
        pltpu.VMEM((128,), jnp.int32),
        pltpu.VMEM((128,), jnp.float32),
        pltpu.VMEM((16,), jnp.int32),
        pltpu.VMEM((16,), jnp.int32),
        pltpu.VMEM((16,), jnp.float32),
        pltpu.VMEM_SHARED((NP,), jnp.float32),
        pltpu.VMEM_SHARED((NP,), jnp.float32),
    ],
)
def _deg_kernel(esrc, edst, zrow, do_out, di_out,
                sbuf, dbuf, iob, iib, onesb, tio, tii, tv, acco, acci):
    cid = lax.axis_index("c")
    sid = lax.axis_index("s")
    tid = cid * 16 + sid
    pltpu.sync_copy(zrow, acco.at[pl.ds(sid * 640, 640)])
    pltpu.sync_copy(zrow, acci.at[pl.ds(sid * 640, 640)])
    onev = jnp.ones((16,), jnp.float32)

    def fill_ones(i, _):
        onesb[pl.ds(i * 16, 16)] = onev
        return 0
    lax.fori_loop(0, 8, fill_ones, 0)
    z16 = jnp.zeros((16,), jnp.int32)
    sbuf[pl.ds(4992, 16)] = z16
    dbuf[pl.ds(4992, 16)] = z16
    base = tid * 5000
    pltpu.sync_copy(esrc.at[pl.ds(base, 5000)], sbuf.at[pl.ds(0, 5000)])
    pltpu.sync_copy(edst.at[pl.ds(base, 5000)], dbuf.at[pl.ds(0, 5000)])
    plsc.subcore_barrier()

    def batch(b, _):
        for j in range(8):
            off = b * 128 + 16 * j
            s16 = sbuf[pl.ds(off, 16)]
            d16 = dbuf[pl.ds(off, 16)]
            iob[pl.ds(16 * j, 16)] = jnp.clip(s16, 0, NP - 1)
            iib[pl.ds(16 * j, 16)] = jnp.clip(d16, 0, NP - 1)
        pltpu.sync_copy(onesb, acco.at[iob], add=True)
        pltpu.sync_copy(onesb, acci.at[iib], add=True)
        return 0
    lax.fori_loop(0, 39, batch, 0)
    valid = _iota16() < 8
    s16 = sbuf[pl.ds(4992, 16)]
    d16 = dbuf[pl.ds(4992, 16)]
    tio[...] = jnp.clip(s16, 0, NP - 1)
    tii[...] = jnp.clip(d16, 0, NP - 1)
    tv[...] = jnp.where(valid, 1.0, 0.0).astype(jnp.float32)
    pltpu.sync_copy(tv, acco.at[tio], add=True)
    pltpu.sync_copy(tv, acci.at[tii], add=True)
    plsc.subcore_barrier()
    pltpu.sync_copy(acco.at[pl.ds(sid * 640, 640)],
                    do_out.at[cid, pl.ds(sid * 640, 640)])
    pltpu.sync_copy(acci.at[pl.ds(sid * 640, 640)],
                    di_out.at[cid, pl.ds(sid * 640, 640)])


NPG = NP + 128


@functools.partial(
    pl.kernel, mesh=_mesh,
    out_type=[jax.ShapeDtypeStruct((2, NPG), jnp.float32)],
    scratch_types=[
        pltpu.VMEM((5008,), jnp.int32),
        pltpu.VMEM((5008,), jnp.int32),
        pltpu.VMEM((128,), jnp.int32),
        pltpu.VMEM((128,), jnp.int32),
        pltpu.VMEM((128,), jnp.float32),
        pltpu.VMEM_SHARED((NPG,), jnp.float32),
        pltpu.SemaphoreType.DMA,
    ],
)
def _cvec_kernel(esrc, edst, ni, zrow, c_out, sbuf, dbuf, ib, db, vb, acc, sem):
    cid = lax.axis_index("c")
    sid = lax.axis_index("s")
    tid = cid * 16 + sid
    pltpu.sync_copy(zrow, acc.at[pl.ds(sid * 640, 640)])

    @pl.when(sid == 0)
    def _():
        pltpu.sync_copy(zrow.at[pl.ds(0, 128)], acc.at[pl.ds(NP, 128)])
    z16 = jnp.zeros((16,), jnp.int32)
    sbuf[pl.ds(4992, 16)] = z16
    dbuf[pl.ds(4992, 16)] = z16
    base = tid * 5000
    pltpu.sync_copy(esrc.at[pl.ds(base, 5000)], sbuf.at[pl.ds(0, 5000)])
    pltpu.sync_copy(edst.at[pl.ds(base, 5000)], dbuf.at[pl.ds(0, 5000)])
    plsc.subcore_barrier()
    iot = _iota16()
    garb = NP + sid * 4 + (iot & 3)

    def batch(b, _):
        for j in range(8):
            off = b * 128 + 16 * j
            s16 = sbuf[pl.ds(off, 16)]
            d16 = dbuf[pl.ds(off, 16)]
            ib[pl.ds(16 * j, 16)] = jnp.clip(s16, 0, NP - 1)
            db[pl.ds(16 * j, 16)] = jnp.clip(d16, 0, NP - 1)
        pltpu.async_copy(ni.at[db], vb, sem).wait()
        pltpu.sync_copy(vb, acc.at[ib], add=True)
        return 0
    lax.fori_loop(0, 39, batch, 0)
    valid = iot < 8
    s16 = jnp.clip(sbuf[pl.ds(4992, 16)], 0, NP - 1)
    d16 = jnp.clip(dbuf[pl.ds(4992, 16)], 0, NP - 1)
    ib[pl.ds(0, 16)] = jnp.where(valid, s16, garb)
    db[pl.ds(0, 16)] = d16

    def padrest(i, _):
        ib[pl.ds(16 + i * 16, 16)] = garb
        db[pl.ds(16 + i * 16, 16)] = z16
        return 0
    lax.fori_loop(0, 7, padrest, 0)
    pltpu.async_copy(ni.at[db], vb, sem).wait()
    pltpu.sync_copy(vb, acc.at[ib], add=True)
    plsc.subcore_barrier()
    pltpu.sync_copy(acc.at[pl.ds(sid * 640, 640)],
                    c_out.at[cid, pl.ds(sid * 640, 640)])

    @pl.when(sid == 0)
    def _():
        pltpu.sync_copy(acc.at[pl.ds(NP, 128)], c_out.at[cid, pl.ds(NP, 128)])


EPT = E // 16


@functools.partial(
    pl.kernel, mesh=_mesh,
    out_type=[jax.ShapeDtypeStruct((NP, 128), jnp.float32),
              jax.ShapeDtypeStruct((NP, 128), jnp.float32)],
    scratch_types=[
        pltpu.VMEM((2048,), jnp.int32),
        pltpu.VMEM((2048,), jnp.int32),
        pltpu.VMEM((64,), jnp.int32),
        pltpu.VMEM((64,), jnp.int32),
        pltpu.VMEM((64,), jnp.int32),
        pltpu.VMEM((64,), jnp.int32),
        pltpu.VMEM((64, 128), jnp.float32),
        pltpu.VMEM((64, 128), jnp.float32),
        pltpu.VMEM((64, 128), jnp.float32),
        pltpu.VMEM((64, 128), jnp.float32),
        pltpu.VMEM_SHARED((ACC_ROWS, 128), jnp.float32),
        pltpu.VMEM_SHARED((ACC_ROWS, 128), jnp.float32),
        pltpu.SemaphoreType.DMA,
        pltpu.SemaphoreType.DMA,
        pltpu.SemaphoreType.DMA,
        pltpu.SemaphoreType.DMA,
    ],
)
def _prop_kernel(xa, xb, esrc, edst, zstripe, ya, yb,
                 sbuf, dbuf, gf0, gf1, df0, df1,
                 sa0, sa1, sb0, sb1, acc_a, acc_b,
                 ma0, ma1, mb0, mb1):
    cid = lax.axis_index("c")
    sid = lax.axis_index("s")
    lo = cid * HALF
    pltpu.sync_copy(zstripe, acc_a.at[pl.ds(sid * 328, 328)])
    pltpu.sync_copy(zstripe, acc_b.at[pl.ds(sid * 328, 328)])
    plsc.subcore_barrier()
    iot = _iota16()
    garb = HALF + sid * 8 + (iot & 7)
    gfs = (gf0, gf1)
    dfs = (df0, df1)

    def build(b, slot):
        for j in range(4):
            off = b * 64 + 16 * j
            s16 = jnp.clip(sbuf[pl.ds(off, 16)], 0, NP - 1)
            d16 = dbuf[pl.ds(off, 16)]
            m = (d16 >= lo) & (d16 < lo + HALF)
            gfs[slot][pl.ds(16 * j, 16)] = s16
            dfs[slot][pl.ds(16 * j, 16)] = jnp.where(m, d16 - lo, garb)

    def pair(p, _):
        build(2 * p, 0)
        cpa0 = pltpu.async_copy(xa.at[gf0], sa0, ma0)
        cpb0 = pltpu.async_copy(xb.at[gf0], sb0, mb0)
        build(2 * p + 1, 1)
        cpa1 = pltpu.async_copy(xa.at[gf1], sa1, ma1)
        cpb1 = pltpu.async_copy(xb.at[gf1], sb1, mb1)
        cpa0.wait()
        pltpu.sync_copy(sa0, acc_a.at[df0], add=True)
        cpb0.wait()
        pltpu.sync_copy(sb0, acc_b.at[df0], add=True)
        cpa1.wait()
        pltpu.sync_copy(sa1, acc_a.at[df1], add=True)
        cpb1.wait()
        pltpu.sync_copy(sb1, acc_b.at[df1], add=True)
        return 0

    def chunk(c, _):
        pltpu.sync_copy(esrc.at[pl.ds(sid * EPT + 2048 * c, 2048)], sbuf)
        pltpu.sync_copy(edst.at[pl.ds(sid * EPT + 2048 * c, 2048)], dbuf)
        lax.fori_loop(0, 16, pair, 0)
        return 0
    lax.fori_loop(0, 4, chunk, 0)
    neg1 = jnp.full((16,), -1, jnp.int32)

    def padtail(k, _):
        sbuf[pl.ds(1808 + k * 16, 16)] = iot + k * 16
        dbuf[pl.ds(1808 + k * 16, 16)] = neg1
        return 0
    lax.fori_loop(0, 15, padtail, 0)
    pltpu.sync_copy(esrc.at[pl.ds(sid * EPT + 8192, 1808)],
                    sbuf.at[pl.ds(0, 1808)])
    pltpu.sync_copy(edst.at[pl.ds(sid * EPT + 8192, 1808)],
                    dbuf.at[pl.ds(0, 1808)])
    lax.fori_loop(0, 16, pair, 0)
    plsc.subcore_barrier()
    row0 = sid * 320
    for k in range(4):
        pltpu.sync_copy(acc_a.at[pl.ds(row0 + 80 * k, 80)],
                        ya.at[pl.ds(cid * HALF + row0 + 80 * k, 80)])
        pltpu.sync_copy(acc_b.at[pl.ds(row0 + 80 * k, 80)],
                        yb.at[pl.ds(cid * HALF + row0 + 80 * k, 80)])


def _scale_body(xf, dop, dip, x0a, x0b, no_o, ni_o):
    do = dop[0] + dop[1]
    di = dip[0] + dip[1]
    no = lax.rsqrt(jnp.clip(do, 1.0, None))
    ni = lax.rsqrt(jnp.clip(di, 1.0, None))
    no_o[...] = no
    ni_o[...] = ni
    x0 = xf[...] * no
    x0a[...] = x0[:, :128]
    x0b[...] = x0[:, 128:]


def _scale(xpad, do3, di3):
    return pl.pallas_call(
        _scale_body,
        grid=(16,),
        in_specs=[
            pl.BlockSpec((640, F), lambda b: (b, 0)),
            pl.BlockSpec((2, 640, 1), lambda b: (0, b, 0)),
            pl.BlockSpec((2, 640, 1), lambda b: (0, b, 0)),
        ],
        out_specs=[
            pl.BlockSpec((640, 128), lambda b: (b, 0)),
            pl.BlockSpec((640, 128), lambda b: (b, 0)),
            pl.BlockSpec((640, 1), lambda b: (b, 0)),
            pl.BlockSpec((640, 1), lambda b: (b, 0)),
        ],
        out_shape=[
            jax.ShapeDtypeStruct((NP, 128), jnp.float32),
            jax.ShapeDtypeStruct((NP, 128), jnp.float32),
            jax.ShapeDtypeStruct((NP, 1), jnp.float32),
            jax.ShapeDtypeStruct((NP, 1), jnp.float32),
        ],
    )(xpad, do3, di3)


def _mlp1_body(p1a, p1b, ni, no, w1, b1, xa, xb, xc, xd):
    nic = ni[...]
    noc = no[...]
    w1full = w1[...]
    h = jnp.dot(p1a[...] * nic, w1full[:128], preferred_element_type=jnp.float32)
    h += jnp.dot(p1b[...] * nic, w1full[128:], preferred_element_type=jnp.float32)
    h = jnp.maximum(h + b1[...], 0.0) * noc
    xa[...] = h[:, :128]
    xb[...] = h[:, 128:256]
    xc[...] = h[:, 256:384]
    xd[...] = h[:, 384:]


def _mlp1(p1a, p1b, ni2, no2, W1, b1):
    return pl.pallas_call(
        _mlp1_body,
        grid=(20,),
        in_specs=[
            pl.BlockSpec((512, 128), lambda b: (b, 0)),
            pl.BlockSpec((512, 128), lambda b: (b, 0)),
            pl.BlockSpec((512, 1), lambda b: (b, 0)),
            pl.BlockSpec((512, 1), lambda b: (b, 0)),
            pl.BlockSpec((F, H), lambda b: (0, 0)),
            pl.BlockSpec((1, H), lambda b: (0, 0)),
        ],
        out_specs=[
            pl.BlockSpec((512, 128), lambda b: (b, 0)),
            pl.BlockSpec((512, 128), lambda b: (b, 0)),
            pl.BlockSpec((512, 128), lambda b: (b, 0)),
            pl.BlockSpec((512, 128), lambda b: (b, 0)),
        ],
        out_shape=[
            jax.ShapeDtypeStruct((NP, 128), jnp.float32),
            jax.ShapeDtypeStruct((NP, 128), jnp.float32),
            jax.ShapeDtypeStruct((NP, 128), jnp.float32),
            jax.ShapeDtypeStruct((NP, 128), jnp.float32),
        ],
    )(p1a, p1b, ni2, no2, W1, b1)


def _mlp2_body(p2a, p2b, p2c, p2d, ni, no, cp, w2, b2, w3, wc, b3, bc, out, accv):
    b = pl.program_id(0)
    nic = ni[...]
    w2full = w2[...]
    h = jnp.dot(p2a[...] * nic, w2full[:128], preferred_element_type=jnp.float32)
    h += jnp.dot(p2b[...] * nic, w2full[128:256], preferred_element_type=jnp.float32)
    h += jnp.dot(p2c[...] * nic, w2full[256:384], preferred_element_type=jnp.float32)
    h += jnp.dot(p2d[...] * nic, w2full[384:], preferred_element_type=jnp.float32)
    h = jnp.maximum(h + b2[...], 0.0)
    c = cp[0] + cp[1]
    w = (c * no[...]) * (1.0 / N)
    v = jnp.sum(h * w, axis=0, keepdims=True)

    @pl.when(b == 0)
    def _():
        accv[...] = jnp.zeros_like(accv)
    accv[...] += v

    @pl.when(b == 19)
    def _():
        hg = jnp.dot(accv[...], w3[...], preferred_element_type=jnp.float32) + b3[...]
        out[...] = jnp.dot(hg, wc[...], preferred_element_type=jnp.float32) + bc[...]


def _mlp2(p2a, p2b, p2c, p2d, ni2, no2, cp3, W2, b2, W3, Wc, b3, bc):
    return pl.pallas_call(
        _mlp2_body,
        grid=(20,),
        in_specs=[
            pl.BlockSpec((512, 128), lambda b: (b, 0)),
            pl.BlockSpec((512, 128), lambda b: (b, 0)),
            pl.BlockSpec((512, 128), lambda b: (b, 0)),
            pl.BlockSpec((512, 128), lambda b: (b, 0)),
            pl.BlockSpec((512, 1), lambda b: (b, 0)),
            pl.BlockSpec((512, 1), lambda b: (b, 0)),
            pl.BlockSpec((2, 512, 1), lambda b: (0, b, 0)),
            pl.BlockSpec((H, H), lambda b: (0, 0)),
            pl.BlockSpec((1, H), lambda b: (0, 0)),
            pl.BlockSpec((H, H), lambda b: (0, 0)),
            pl.BlockSpec((H, NCLS), lambda b: (0, 0)),
            pl.BlockSpec((1, H), lambda b: (0, 0)),
            pl.BlockSpec((1, NCLS), lambda b: (0, 0)),
        ],
        out_specs=pl.BlockSpec((1, NCLS), lambda b: (0, 0)),
        out_shape=jax.ShapeDtypeStruct((1, NCLS), jnp.float32),
        scratch_shapes=[pltpu.VMEM((1, H), jnp.float32)],
    )(p2a, p2b, p2c, p2d, ni2, no2, cp3, W2, b2, W3, Wc, b3, bc)


def kernel(in_feat, edge_index, W1, b1, W2, b2, W3, b3, Wc, bc):
    f32 = jnp.float32
    xpad = jnp.pad(in_feat, ((0, NP - N), (0, 0)))
    zrow = jnp.zeros((640,), f32)
    zstripe = jnp.zeros((328, 128), f32)
    esrc = edge_index[0]
    edst = edge_index[1]

    do_p, di_p = _deg_kernel(esrc, edst, zrow)
    do3 = do_p.reshape(2, NP, 1)
    di3 = di_p.reshape(2, NP, 1)
    x0a, x0b, no2, ni2 = _scale(xpad, do3, di3)
    (c_pg,) = _cvec_kernel(esrc, edst, ni2.reshape(NP), zrow)
    c_p = c_pg[:, :NP]
    p1a, p1b = _prop_kernel(x0a, x0b, esrc, edst, zstripe)
    x1a, x1b, x1c, x1d = _mlp1(p1a, p1b, ni2, no2, W1, b1.reshape(1, H))
    p2a, p2b = _prop_kernel(x1a, x1b, esrc, edst, zstripe)
    p2c, p2d = _prop_kernel(x1c, x1d, esrc, edst, zstripe)
    out = _mlp2(p2a, p2b, p2c, p2d, ni2, no2, c_p.reshape(2, NP, 1),
                W2, b2.reshape(1, H), W3, Wc, b3.reshape(1, H),
                bc.reshape(1, NCLS))
    return out


# speedup vs baseline: 6.9071x; 1.5640x over previous
import functools

import jax
import jax.numpy as jnp
from jax import lax
from jax.experimental import pallas as pl
from jax.experimental.pallas import tpu as pltpu
from jax.experimental.pallas import tpu_sc as plsc

N = 10000
E = 160000
F = 256
H = 512
NP = 10240
HALF = NP // 2
ACC_ROWS = HALF + 128
NCLS = 10

_mesh = plsc.VectorSubcoreMesh(core_axis_name="c", subcore_axis_name="s")


def _iota16():
    return lax.iota(jnp.int32, 16)


@functools.partial(
    pl.kernel, mesh=_mesh,
    out_type=[jax.ShapeDtypeStruct((2, NP), jnp.float32),
              jax.ShapeDtypeStruct((2, NP), jnp.float32)],
    scratch_types=[
        pltpu.VMEM((5008,), jnp.int32),
        pltpu.VMEM((5008,), jnp.int32),
        pltpu.VMEM((128,), jnp.int32),
        pltpu.VMEM((128,), jnp.int32),
        pltpu.VMEM((128,), jnp.float32),
        pltpu.VMEM((16,), jnp.int32),
        pltpu.VMEM((16,), jnp.int32),
        pltpu.VMEM((16,), jnp.float32),
        pltpu.VMEM_SHARED((NP,), jnp.float32),
        pltpu.VMEM_SHARED((NP,), jnp.float32),
    ],
)
def _deg_kernel(esrc, edst, zrow, do_out, di_out,
                sbuf, dbuf, iob, iib, onesb, tio, tii, tv, acco, acci):
    cid = lax.axis_index("c")
    sid = lax.axis_index("s")
    tid = cid * 16 + sid
    pltpu.sync_copy(zrow, acco.at[pl.ds(sid * 640, 640)])
    pltpu.sync_copy(zrow, acci.at[pl.ds(sid * 640, 640)])
    onev = jnp.ones((16,), jnp.float32)

    def fill_ones(i, _):
        onesb[pl.ds(i * 16, 16)] = onev
        return 0
    lax.fori_loop(0, 8, fill_ones, 0)
    z16 = jnp.zeros((16,), jnp.int32)
    sbuf[pl.ds(4992, 16)] = z16
    dbuf[pl.ds(4992, 16)] = z16
    base = tid * 5000
    pltpu.sync_copy(esrc.at[pl.ds(base, 5000)], sbuf.at[pl.ds(0, 5000)])
    pltpu.sync_copy(edst.at[pl.ds(base, 5000)], dbuf.at[pl.ds(0, 5000)])
    plsc.subcore_barrier()

    def batch(b, _):
        for j in range(8):
            off = b * 128 + 16 * j
            s16 = sbuf[pl.ds(off, 16)]
            d16 = dbuf[pl.ds(off, 16)]
            iob[pl.ds(16 * j, 16)] = jnp.clip(s16, 0, NP - 1)
            iib[pl.ds(16 * j, 16)] = jnp.clip(d16, 0, NP - 1)
        pltpu.sync_copy(onesb, acco.at[iob], add=True)
        pltpu.sync_copy(onesb, acci.at[iib], add=True)
        return 0
    lax.fori_loop(0, 39, batch, 0)
    valid = _iota16() < 8
    s16 = sbuf[pl.ds(4992, 16)]
    d16 = dbuf[pl.ds(4992, 16)]
    tio[...] = jnp.clip(s16, 0, NP - 1)
    tii[...] = jnp.clip(d16, 0, NP - 1)
    tv[...] = jnp.where(valid, 1.0, 0.0).astype(jnp.float32)
    pltpu.sync_copy(tv, acco.at[tio], add=True)
    pltpu.sync_copy(tv, acci.at[tii], add=True)
    plsc.subcore_barrier()
    pltpu.sync_copy(acco.at[pl.ds(sid * 640, 640)],
                    do_out.at[cid, pl.ds(sid * 640, 640)])
    pltpu.sync_copy(acci.at[pl.ds(sid * 640, 640)],
                    di_out.at[cid, pl.ds(sid * 640, 640)])


NPG = NP + 128


@functools.partial(
    pl.kernel, mesh=_mesh,
    out_type=[jax.ShapeDtypeStruct((2, NPG), jnp.float32)],
    scratch_types=[
        pltpu.VMEM((5008,), jnp.int32),
        pltpu.VMEM((5008,), jnp.int32),
        pltpu.VMEM((128,), jnp.int32),
        pltpu.VMEM((128,), jnp.int32),
        pltpu.VMEM((128,), jnp.float32),
        pltpu.VMEM_SHARED((NPG,), jnp.float32),
        pltpu.SemaphoreType.DMA,
    ],
)
def _cvec_kernel(esrc, edst, ni, zrow, c_out, sbuf, dbuf, ib, db, vb, acc, sem):
    cid = lax.axis_index("c")
    sid = lax.axis_index("s")
    tid = cid * 16 + sid
    pltpu.sync_copy(zrow, acc.at[pl.ds(sid * 640, 640)])

    @pl.when(sid == 0)
    def _():
        pltpu.sync_copy(zrow.at[pl.ds(0, 128)], acc.at[pl.ds(NP, 128)])
    z16 = jnp.zeros((16,), jnp.int32)
    sbuf[pl.ds(4992, 16)] = z16
    dbuf[pl.ds(4992, 16)] = z16
    base = tid * 5000
    pltpu.sync_copy(esrc.at[pl.ds(base, 5000)], sbuf.at[pl.ds(0, 5000)])
    pltpu.sync_copy(edst.at[pl.ds(base, 5000)], dbuf.at[pl.ds(0, 5000)])
    plsc.subcore_barrier()
    iot = _iota16()
    garb = NP + sid * 4 + (iot & 3)

    def batch(b, _):
        for j in range(8):
            off = b * 128 + 16 * j
            s16 = sbuf[pl.ds(off, 16)]
            d16 = dbuf[pl.ds(off, 16)]
            ib[pl.ds(16 * j, 16)] = jnp.clip(s16, 0, NP - 1)
            db[pl.ds(16 * j, 16)] = jnp.clip(d16, 0, NP - 1)
        pltpu.async_copy(ni.at[db], vb, sem).wait()
        pltpu.sync_copy(vb, acc.at[ib], add=True)
        return 0
    lax.fori_loop(0, 39, batch, 0)
    valid = iot < 8
    s16 = jnp.clip(sbuf[pl.ds(4992, 16)], 0, NP - 1)
    d16 = jnp.clip(dbuf[pl.ds(4992, 16)], 0, NP - 1)
    ib[pl.ds(0, 16)] = jnp.where(valid, s16, garb)
    db[pl.ds(0, 16)] = d16

    def padrest(i, _):
        ib[pl.ds(16 + i * 16, 16)] = garb
        db[pl.ds(16 + i * 16, 16)] = z16
        return 0
    lax.fori_loop(0, 7, padrest, 0)
    pltpu.async_copy(ni.at[db], vb, sem).wait()
    pltpu.sync_copy(vb, acc.at[ib], add=True)
    plsc.subcore_barrier()
    pltpu.sync_copy(acc.at[pl.ds(sid * 640, 640)],
                    c_out.at[cid, pl.ds(sid * 640, 640)])

    @pl.when(sid == 0)
    def _():
        pltpu.sync_copy(acc.at[pl.ds(NP, 128)], c_out.at[cid, pl.ds(NP, 128)])


EPT = E // 16
NPR = NP + 128


@functools.partial(
    pl.kernel, mesh=_mesh,
    out_type=[jax.ShapeDtypeStruct((NP, 128), jnp.float32),
              jax.ShapeDtypeStruct((NP, 128), jnp.float32)],
    scratch_types=[
        pltpu.VMEM((2048,), jnp.int32),
        pltpu.VMEM((2048,), jnp.int32),
        pltpu.VMEM((128,), jnp.int32),
        pltpu.VMEM((128,), jnp.int32),
        pltpu.VMEM((128,), jnp.int32),
        pltpu.VMEM((128,), jnp.int32),
        pltpu.VMEM((128, 128), jnp.float32),
        pltpu.VMEM((128, 128), jnp.float32),
        pltpu.VMEM_SHARED((NPR, 128), jnp.float32),
        pltpu.SemaphoreType.DMA,
        pltpu.SemaphoreType.DMA,
    ],
)
def _prop_kernel(xa, xb, esrc, edst, zstripe, ya, yb,
                 sbuf, dbuf, gf0, gf1, df0, df1,
                 sa0, sa1, acc, ma0, ma1):
    cid = lax.axis_index("c")
    sid = lax.axis_index("s")
    pltpu.sync_copy(zstripe, acc.at[pl.ds(sid * 648, 648)])
    plsc.subcore_barrier()
    iot = _iota16()
    garb = NP + sid * 8 + (iot & 7)
    neg1 = jnp.full((16,), -1, jnp.int32)
    gfs = (gf0, gf1)
    dfs = (df0, df1)
    sas = (sa0, sa1)
    mas = (ma0, ma1)

    def build(b, slot):
        for j in range(8):
            off = b * 128 + 16 * j
            s16 = jnp.clip(sbuf[pl.ds(off, 16)], 0, NP - 1)
            d16 = jnp.clip(dbuf[pl.ds(off, 16)], 0, NPR - 1)
            gfs[slot][pl.ds(16 * j, 16)] = s16
            dfs[slot][pl.ds(16 * j, 16)] = d16

    def scan(xref):
        def pair(p, _):
            build(2 * p, 0)
            cp0 = pltpu.async_copy(xref.at[gf0], sa0, ma0)
            build(2 * p + 1, 1)
            cp1 = pltpu.async_copy(xref.at[gf1], sa1, ma1)
            cp0.wait()
            pltpu.sync_copy(sa0, acc.at[df0], add=True)
            cp1.wait()
            pltpu.sync_copy(sa1, acc.at[df1], add=True)
            return 0

        def chunk(c, _):
            pltpu.sync_copy(esrc.at[pl.ds(sid * EPT + 2048 * c, 2048)], sbuf)
            pltpu.sync_copy(edst.at[pl.ds(sid * EPT + 2048 * c, 2048)], dbuf)
            lax.fori_loop(0, 8, pair, 0)
            return 0
        lax.fori_loop(0, 4, chunk, 0)

        def padtail(k, _):
            sbuf[pl.ds(1808 + k * 16, 16)] = iot
            dbuf[pl.ds(1808 + k * 16, 16)] = garb
            return 0
        lax.fori_loop(0, 15, padtail, 0)
        pltpu.sync_copy(esrc.at[pl.ds(sid * EPT + 8192, 1808)],
                        sbuf.at[pl.ds(0, 1808)])
        pltpu.sync_copy(edst.at[pl.ds(sid * EPT + 8192, 1808)],
                        dbuf.at[pl.ds(0, 1808)])
        lax.fori_loop(0, 8, pair, 0)

    @pl.when(cid == 0)
    def _():
        scan(xa)

    @pl.when(cid == 1)
    def _():
        scan(xb)
    plsc.subcore_barrier()
    row0 = sid * 640

    @pl.when(cid == 0)
    def _():
        for k in range(2):
            pltpu.sync_copy(acc.at[pl.ds(row0 + 320 * k, 320)],
                            ya.at[pl.ds(row0 + 320 * k, 320)])

    @pl.when(cid == 1)
    def _():
        for k in range(2):
            pltpu.sync_copy(acc.at[pl.ds(row0 + 320 * k, 320)],
                            yb.at[pl.ds(row0 + 320 * k, 320)])


def _scale_body(xf, dop, dip, x0a, x0b, no_o, ni_o):
    do = dop[0] + dop[1]
    di = dip[0] + dip[1]
    no = lax.rsqrt(jnp.clip(do, 1.0, None))
    ni = lax.rsqrt(jnp.clip(di, 1.0, None))
    no_o[...] = no
    ni_o[...] = ni
    x0 = xf[...] * no
    x0a[...] = x0[:, :128]
    x0b[...] = x0[:, 128:]


def _scale(xpad, do3, di3):
    return pl.pallas_call(
        _scale_body,
        grid=(16,),
        in_specs=[
            pl.BlockSpec((640, F), lambda b: (b, 0)),
            pl.BlockSpec((2, 640, 1), lambda b: (0, b, 0)),
            pl.BlockSpec((2, 640, 1), lambda b: (0, b, 0)),
        ],
        out_specs=[
            pl.BlockSpec((640, 128), lambda b: (b, 0)),
            pl.BlockSpec((640, 128), lambda b: (b, 0)),
            pl.BlockSpec((640, 1), lambda b: (b, 0)),
            pl.BlockSpec((640, 1), lambda b: (b, 0)),
        ],
        out_shape=[
            jax.ShapeDtypeStruct((NP, 128), jnp.float32),
            jax.ShapeDtypeStruct((NP, 128), jnp.float32),
            jax.ShapeDtypeStruct((NP, 1), jnp.float32),
            jax.ShapeDtypeStruct((NP, 1), jnp.float32),
        ],
    )(xpad, do3, di3)


def _mlp1_body(p1a, p1b, ni, no, w1, b1, xa, xb, xc, xd):
    nic = ni[...]
    noc = no[...]
    w1full = w1[...]
    h = jnp.dot(p1a[...] * nic, w1full[:128], preferred_element_type=jnp.float32)
    h += jnp.dot(p1b[...] * nic, w1full[128:], preferred_element_type=jnp.float32)
    h = jnp.maximum(h + b1[...], 0.0) * noc
    xa[...] = h[:, :128]
    xb[...] = h[:, 128:256]
    xc[...] = h[:, 256:384]
    xd[...] = h[:, 384:]


def _mlp1(p1a, p1b, ni2, no2, W1, b1):
    return pl.pallas_call(
        _mlp1_body,
        grid=(20,),
        in_specs=[
            pl.BlockSpec((512, 128), lambda b: (b, 0)),
            pl.BlockSpec((512, 128), lambda b: (b, 0)),
            pl.BlockSpec((512, 1), lambda b: (b, 0)),
            pl.BlockSpec((512, 1), lambda b: (b, 0)),
            pl.BlockSpec((F, H), lambda b: (0, 0)),
            pl.BlockSpec((1, H), lambda b: (0, 0)),
        ],
        out_specs=[
            pl.BlockSpec((512, 128), lambda b: (b, 0)),
            pl.BlockSpec((512, 128), lambda b: (b, 0)),
            pl.BlockSpec((512, 128), lambda b: (b, 0)),
            pl.BlockSpec((512, 128), lambda b: (b, 0)),
        ],
        out_shape=[
            jax.ShapeDtypeStruct((NP, 128), jnp.float32),
            jax.ShapeDtypeStruct((NP, 128), jnp.float32),
            jax.ShapeDtypeStruct((NP, 128), jnp.float32),
            jax.ShapeDtypeStruct((NP, 128), jnp.float32),
        ],
    )(p1a, p1b, ni2, no2, W1, b1)


def _mlp2_body(p2a, p2b, p2c, p2d, ni, no, cp, w2, b2, w3, wc, b3, bc, out, accv):
    b = pl.program_id(0)
    nic = ni[...]
    w2full = w2[...]
    h = jnp.dot(p2a[...] * nic, w2full[:128], preferred_element_type=jnp.float32)
    h += jnp.dot(p2b[...] * nic, w2full[128:256], preferred_element_type=jnp.float32)
    h += jnp.dot(p2c[...] * nic, w2full[256:384], preferred_element_type=jnp.float32)
    h += jnp.dot(p2d[...] * nic, w2full[384:], preferred_element_type=jnp.float32)
    h = jnp.maximum(h + b2[...], 0.0)
    c = cp[0] + cp[1]
    w = (c * no[...]) * (1.0 / N)
    v = jnp.sum(h * w, axis=0, keepdims=True)

    @pl.when(b == 0)
    def _():
        accv[...] = jnp.zeros_like(accv)
    accv[...] += v

    @pl.when(b == 19)
    def _():
        hg = jnp.dot(accv[...], w3[...], preferred_element_type=jnp.float32) + b3[...]
        out[...] = jnp.dot(hg, wc[...], preferred_element_type=jnp.float32) + bc[...]


def _mlp2(p2a, p2b, p2c, p2d, ni2, no2, cp3, W2, b2, W3, Wc, b3, bc):
    return pl.pallas_call(
        _mlp2_body,
        grid=(20,),
        in_specs=[
            pl.BlockSpec((512, 128), lambda b: (b, 0)),
            pl.BlockSpec((512, 128), lambda b: (b, 0)),
            pl.BlockSpec((512, 128), lambda b: (b, 0)),
            pl.BlockSpec((512, 128), lambda b: (b, 0)),
            pl.BlockSpec((512, 1), lambda b: (b, 0)),
            pl.BlockSpec((512, 1), lambda b: (b, 0)),
            pl.BlockSpec((2, 512, 1), lambda b: (0, b, 0)),
            pl.BlockSpec((H, H), lambda b: (0, 0)),
            pl.BlockSpec((1, H), lambda b: (0, 0)),
            pl.BlockSpec((H, H), lambda b: (0, 0)),
            pl.BlockSpec((H, NCLS), lambda b: (0, 0)),
            pl.BlockSpec((1, H), lambda b: (0, 0)),
            pl.BlockSpec((1, NCLS), lambda b: (0, 0)),
        ],
        out_specs=pl.BlockSpec((1, NCLS), lambda b: (0, 0)),
        out_shape=jax.ShapeDtypeStruct((1, NCLS), jnp.float32),
        scratch_shapes=[pltpu.VMEM((1, H), jnp.float32)],
    )(p2a, p2b, p2c, p2d, ni2, no2, cp3, W2, b2, W3, Wc, b3, bc)


def kernel(in_feat, edge_index, W1, b1, W2, b2, W3, b3, Wc, bc):
    f32 = jnp.float32
    xpad = jnp.pad(in_feat, ((0, NP - N), (0, 0)))
    zrow = jnp.zeros((640,), f32)
    zstripe = jnp.zeros((648, 128), f32)
    esrc = edge_index[0]
    edst = edge_index[1]

    do_p, di_p = _deg_kernel(esrc, edst, zrow)
    do3 = do_p.reshape(2, NP, 1)
    di3 = di_p.reshape(2, NP, 1)
    x0a, x0b, no2, ni2 = _scale(xpad, do3, di3)
    (c_pg,) = _cvec_kernel(esrc, edst, ni2.reshape(NP), zrow)
    c_p = c_pg[:, :NP]
    p1a, p1b = _prop_kernel(x0a, x0b, esrc, edst, zstripe)
    x1a, x1b, x1c, x1d = _mlp1(p1a, p1b, ni2, no2, W1, b1.reshape(1, H))
    p2a, p2b = _prop_kernel(x1a, x1b, esrc, edst, zstripe)
    p2c, p2d = _prop_kernel(x1c, x1d, esrc, edst, zstripe)
    out = _mlp2(p2a, p2b, p2c, p2d, ni2, no2, c_p.reshape(2, NP, 1),
                W2, b2.reshape(1, H), W3, Wc, b3.reshape(1, H),
                bc.reshape(1, NCLS))
    return out


# speedup vs baseline: 7.2939x; 1.0560x over previous
import functools

import jax
import jax.numpy as jnp
from jax import lax
from jax.experimental import pallas as pl
from jax.experimental.pallas import tpu as pltpu
from jax.experimental.pallas import tpu_sc as plsc

N = 10000
E = 160000
F = 256
H = 512
NP = 10240
HALF = NP // 2
ACC_ROWS = HALF + 128
NCLS = 10

_mesh = plsc.VectorSubcoreMesh(core_axis_name="c", subcore_axis_name="s")


def _iota16():
    return lax.iota(jnp.int32, 16)


@functools.partial(
    pl.kernel, mesh=_mesh,
    out_type=[jax.ShapeDtypeStruct((2, NP), jnp.float32),
              jax.ShapeDtypeStruct((2, NP), jnp.float32)],
    scratch_types=[
        pltpu.VMEM((5008,), jnp.int32),
        pltpu.VMEM((5008,), jnp.int32),
        pltpu.VMEM((128,), jnp.int32),
        pltpu.VMEM((128,), jnp.int32),
        pltpu.VMEM((128,), jnp.float32),
        pltpu.VMEM((16,), jnp.int32),
        pltpu.VMEM((16,), jnp.int32),
        pltpu.VMEM((16,), jnp.float32),
        pltpu.VMEM_SHARED((NP,), jnp.float32),
        pltpu.VMEM_SHARED((NP,), jnp.float32),
    ],
)
def _deg_kernel(esrc, edst, zrow, do_out, di_out,
                sbuf, dbuf, iob, iib, onesb, tio, tii, tv, acco, acci):
    cid = lax.axis_index("c")
    sid = lax.axis_index("s")
    tid = cid * 16 + sid
    pltpu.sync_copy(zrow, acco.at[pl.ds(sid * 640, 640)])
    pltpu.sync_copy(zrow, acci.at[pl.ds(sid * 640, 640)])
    onev = jnp.ones((16,), jnp.float32)

    def fill_ones(i, _):
        onesb[pl.ds(i * 16, 16)] = onev
        return 0
    lax.fori_loop(0, 8, fill_ones, 0)
    z16 = jnp.zeros((16,), jnp.int32)
    sbuf[pl.ds(4992, 16)] = z16
    dbuf[pl.ds(4992, 16)] = z16
    base = tid * 5000
    pltpu.sync_copy(esrc.at[pl.ds(base, 5000)], sbuf.at[pl.ds(0, 5000)])
    pltpu.sync_copy(edst.at[pl.ds(base, 5000)], dbuf.at[pl.ds(0, 5000)])
    plsc.subcore_barrier()

    def batch(b, _):
        for j in range(8):
            off = b * 128 + 16 * j
            s16 = sbuf[pl.ds(off, 16)]
            d16 = dbuf[pl.ds(off, 16)]
            iob[pl.ds(16 * j, 16)] = jnp.clip(s16, 0, NP - 1)
            iib[pl.ds(16 * j, 16)] = jnp.clip(d16, 0, NP - 1)
        pltpu.sync_copy(onesb, acco.at[iob], add=True)
        pltpu.sync_copy(onesb, acci.at[iib], add=True)
        return 0
    lax.fori_loop(0, 39, batch, 0)
    valid = _iota16() < 8
    s16 = sbuf[pl.ds(4992, 16)]
    d16 = dbuf[pl.ds(4992, 16)]
    tio[...] = jnp.clip(s16, 0, NP - 1)
    tii[...] = jnp.clip(d16, 0, NP - 1)
    tv[...] = jnp.where(valid, 1.0, 0.0).astype(jnp.float32)
    pltpu.sync_copy(tv, acco.at[tio], add=True)
    pltpu.sync_copy(tv, acci.at[tii], add=True)
    plsc.subcore_barrier()
    pltpu.sync_copy(acco.at[pl.ds(sid * 640, 640)],
                    do_out.at[cid, pl.ds(sid * 640, 640)])
    pltpu.sync_copy(acci.at[pl.ds(sid * 640, 640)],
                    di_out.at[cid, pl.ds(sid * 640, 640)])


NPG = NP + 128


@functools.partial(
    pl.kernel, mesh=_mesh,
    out_type=[jax.ShapeDtypeStruct((2, NPG), jnp.float32)],
    scratch_types=[
        pltpu.VMEM((5120,), jnp.int32),
        pltpu.VMEM((5120,), jnp.int32),
        pltpu.VMEM((128,), jnp.int32),
        pltpu.VMEM((128,), jnp.int32),
        pltpu.VMEM((128,), jnp.int32),
        pltpu.VMEM((128,), jnp.int32),
        pltpu.VMEM((128,), jnp.float32),
        pltpu.VMEM((128,), jnp.float32),
        pltpu.VMEM_SHARED((NPG,), jnp.float32),
        pltpu.SemaphoreType.DMA,
        pltpu.SemaphoreType.DMA,
    ],
)
def _cvec_kernel(esrc, edst, ni, zrow, c_out,
                 sbuf, dbuf, ib0, ib1, db0, db1, vb0, vb1, acc, sem0, sem1):
    cid = lax.axis_index("c")
    sid = lax.axis_index("s")
    tid = cid * 16 + sid
    pltpu.sync_copy(zrow, acc.at[pl.ds(sid * 640, 640)])

    @pl.when(sid == 0)
    def _():
        pltpu.sync_copy(zrow.at[pl.ds(0, 128)], acc.at[pl.ds(NP, 128)])
    iot = _iota16()
    garb = NP + sid * 4 + (iot & 3)
    z16 = jnp.zeros((16,), jnp.int32)

    def padtail(k, _):
        sbuf[pl.ds(5008 + k * 16, 16)] = garb
        dbuf[pl.ds(5008 + k * 16, 16)] = z16
        return 0
    base = tid * 5000
    pltpu.sync_copy(esrc.at[pl.ds(base, 5000)], sbuf.at[pl.ds(0, 5000)])
    pltpu.sync_copy(edst.at[pl.ds(base, 5000)], dbuf.at[pl.ds(0, 5000)])
    sreal = sbuf[pl.ds(4992, 16)]
    dreal = dbuf[pl.ds(4992, 16)]
    mreal = iot < 8
    sbuf[pl.ds(4992, 16)] = jnp.where(mreal, sreal, garb)
    dbuf[pl.ds(4992, 16)] = jnp.where(mreal, dreal, z16)
    lax.fori_loop(0, 7, padtail, 0)
    plsc.subcore_barrier()
    ibs = (ib0, ib1)
    dbs = (db0, db1)
    vbs = (vb0, vb1)
    sems = (sem0, sem1)

    def build(b, slot):
        for j in range(8):
            off = b * 128 + 16 * j
            s16 = jnp.clip(sbuf[pl.ds(off, 16)], 0, NPG - 1)
            d16 = jnp.clip(dbuf[pl.ds(off, 16)], 0, NP - 1)
            ibs[slot][pl.ds(16 * j, 16)] = s16
            dbs[slot][pl.ds(16 * j, 16)] = d16

    def pair(p, _):
        build(2 * p, 0)
        g0 = pltpu.async_copy(ni.at[db0], vb0, sem0)
        build(2 * p + 1, 1)
        g1 = pltpu.async_copy(ni.at[db1], vb1, sem1)
        g0.wait()
        pltpu.sync_copy(vb0, acc.at[ib0], add=True)
        g1.wait()
        pltpu.sync_copy(vb1, acc.at[ib1], add=True)
        return 0
    lax.fori_loop(0, 20, pair, 0)
    plsc.subcore_barrier()
    pltpu.sync_copy(acc.at[pl.ds(sid * 640, 640)],
                    c_out.at[cid, pl.ds(sid * 640, 640)])

    @pl.when(sid == 0)
    def _():
        pltpu.sync_copy(acc.at[pl.ds(NP, 128)], c_out.at[cid, pl.ds(NP, 128)])


EPT = E // 16
NPR = NP + 128


@functools.partial(
    pl.kernel, mesh=_mesh,
    out_type=[jax.ShapeDtypeStruct((NP, 128), jnp.float32),
              jax.ShapeDtypeStruct((NP, 128), jnp.float32)],
    scratch_types=[
        pltpu.VMEM((2048,), jnp.int32),
        pltpu.VMEM((2048,), jnp.int32),
        pltpu.VMEM((128,), jnp.int32),
        pltpu.VMEM((128,), jnp.int32),
        pltpu.VMEM((128,), jnp.int32),
        pltpu.VMEM((128,), jnp.int32),
        pltpu.VMEM((128, 128), jnp.float32),
        pltpu.VMEM((128, 128), jnp.float32),
        pltpu.VMEM_SHARED((NPR, 128), jnp.float32),
        pltpu.SemaphoreType.DMA,
        pltpu.SemaphoreType.DMA,
        pltpu.SemaphoreType.DMA,
        pltpu.SemaphoreType.DMA,
    ],
)
def _prop_kernel(xa, xb, esrc, edst, zstripe, ya, yb,
                 sbuf, dbuf, gf0, gf1, df0, df1,
                 sa0, sa1, acc, ma0, ma1, wa0, wa1):
    cid = lax.axis_index("c")
    sid = lax.axis_index("s")
    pltpu.sync_copy(zstripe, acc.at[pl.ds(sid * 648, 648)])
    plsc.subcore_barrier()
    iot = _iota16()
    garb = NP + sid * 8 + (iot & 7)
    neg1 = jnp.full((16,), -1, jnp.int32)
    gfs = (gf0, gf1)
    dfs = (df0, df1)
    sas = (sa0, sa1)
    mas = (ma0, ma1)

    def build(b, slot):
        for j in range(8):
            off = b * 128 + 16 * j
            s16 = jnp.clip(sbuf[pl.ds(off, 16)], 0, NP - 1)
            d16 = jnp.clip(dbuf[pl.ds(off, 16)], 0, NPR - 1)
            gfs[slot][pl.ds(16 * j, 16)] = s16
            dfs[slot][pl.ds(16 * j, 16)] = d16

    def scan(xref):
        def wait_scat(slot):
            sref = (sa0, sa1)[slot]
            dref = (df0, df1)[slot]
            wref = (wa0, wa1)[slot]
            pltpu.make_async_copy(sref, acc.at[dref], wref).wait()

        def pair(p, _):
            @pl.when(p > 0)
            def _():
                wait_scat(0)
            build(2 * p, 0)
            cp0 = pltpu.async_copy(xref.at[gf0], sa0, ma0)

            @pl.when(p > 0)
            def _():
                wait_scat(1)
            build(2 * p + 1, 1)
            cp1 = pltpu.async_copy(xref.at[gf1], sa1, ma1)
            cp0.wait()
            pltpu.async_copy(sa0, acc.at[df0], wa0, add=True)
            cp1.wait()
            pltpu.async_copy(sa1, acc.at[df1], wa1, add=True)
            return 0

        def chunk(c, _):
            pltpu.sync_copy(esrc.at[pl.ds(sid * EPT + 2048 * c, 2048)], sbuf)
            pltpu.sync_copy(edst.at[pl.ds(sid * EPT + 2048 * c, 2048)], dbuf)

            @pl.when(c > 0)
            def _():
                wait_scat(0)
                wait_scat(1)
            lax.fori_loop(0, 8, pair, 0)
            return 0
        lax.fori_loop(0, 4, chunk, 0)
        wait_scat(0)
        wait_scat(1)

        def padtail(k, _):
            sbuf[pl.ds(1808 + k * 16, 16)] = iot
            dbuf[pl.ds(1808 + k * 16, 16)] = garb
            return 0
        lax.fori_loop(0, 15, padtail, 0)
        pltpu.sync_copy(esrc.at[pl.ds(sid * EPT + 8192, 1808)],
                        sbuf.at[pl.ds(0, 1808)])
        pltpu.sync_copy(edst.at[pl.ds(sid * EPT + 8192, 1808)],
                        dbuf.at[pl.ds(0, 1808)])
        lax.fori_loop(0, 8, pair, 0)
        wait_scat(0)
        wait_scat(1)

    @pl.when(cid == 0)
    def _():
        scan(xa)

    @pl.when(cid == 1)
    def _():
        scan(xb)
    plsc.subcore_barrier()
    row0 = sid * 640

    @pl.when(cid == 0)
    def _():
        for k in range(2):
            pltpu.sync_copy(acc.at[pl.ds(row0 + 320 * k, 320)],
                            ya.at[pl.ds(row0 + 320 * k, 320)])

    @pl.when(cid == 1)
    def _():
        for k in range(2):
            pltpu.sync_copy(acc.at[pl.ds(row0 + 320 * k, 320)],
                            yb.at[pl.ds(row0 + 320 * k, 320)])


def _scale_body(xf, dop, dip, x0a, x0b, no_o, ni_o):
    do = dop[0] + dop[1]
    di = dip[0] + dip[1]
    no = lax.rsqrt(jnp.clip(do, 1.0, None))
    ni = lax.rsqrt(jnp.clip(di, 1.0, None))
    no_o[...] = no
    ni_o[...] = ni
    x0 = xf[...] * no
    x0a[...] = x0[:, :128]
    x0b[...] = x0[:, 128:]


def _scale(xpad, do3, di3):
    return pl.pallas_call(
        _scale_body,
        grid=(16,),
        in_specs=[
            pl.BlockSpec((640, F), lambda b: (b, 0)),
            pl.BlockSpec((2, 640, 1), lambda b: (0, b, 0)),
            pl.BlockSpec((2, 640, 1), lambda b: (0, b, 0)),
        ],
        out_specs=[
            pl.BlockSpec((640, 128), lambda b: (b, 0)),
            pl.BlockSpec((640, 128), lambda b: (b, 0)),
            pl.BlockSpec((640, 1), lambda b: (b, 0)),
            pl.BlockSpec((640, 1), lambda b: (b, 0)),
        ],
        out_shape=[
            jax.ShapeDtypeStruct((NP, 128), jnp.float32),
            jax.ShapeDtypeStruct((NP, 128), jnp.float32),
            jax.ShapeDtypeStruct((NP, 1), jnp.float32),
            jax.ShapeDtypeStruct((NP, 1), jnp.float32),
        ],
    )(xpad, do3, di3)


def _mlp1_body(p1a, p1b, ni, no, w1, b1, xa, xb, xc, xd):
    nic = ni[...]
    noc = no[...]
    w1full = w1[...]
    h = jnp.dot(p1a[...] * nic, w1full[:128], preferred_element_type=jnp.float32)
    h += jnp.dot(p1b[...] * nic, w1full[128:], preferred_element_type=jnp.float32)
    h = jnp.maximum(h + b1[...], 0.0) * noc
    xa[...] = h[:, :128]
    xb[...] = h[:, 128:256]
    xc[...] = h[:, 256:384]
    xd[...] = h[:, 384:]


def _mlp1(p1a, p1b, ni2, no2, W1, b1):
    return pl.pallas_call(
        _mlp1_body,
        grid=(20,),
        in_specs=[
            pl.BlockSpec((512, 128), lambda b: (b, 0)),
            pl.BlockSpec((512, 128), lambda b: (b, 0)),
            pl.BlockSpec((512, 1), lambda b: (b, 0)),
            pl.BlockSpec((512, 1), lambda b: (b, 0)),
            pl.BlockSpec((F, H), lambda b: (0, 0)),
            pl.BlockSpec((1, H), lambda b: (0, 0)),
        ],
        out_specs=[
            pl.BlockSpec((512, 128), lambda b: (b, 0)),
            pl.BlockSpec((512, 128), lambda b: (b, 0)),
            pl.BlockSpec((512, 128), lambda b: (b, 0)),
            pl.BlockSpec((512, 128), lambda b: (b, 0)),
        ],
        out_shape=[
            jax.ShapeDtypeStruct((NP, 128), jnp.float32),
            jax.ShapeDtypeStruct((NP, 128), jnp.float32),
            jax.ShapeDtypeStruct((NP, 128), jnp.float32),
            jax.ShapeDtypeStruct((NP, 128), jnp.float32),
        ],
    )(p1a, p1b, ni2, no2, W1, b1)


def _mlp2_body(p2a, p2b, p2c, p2d, ni, no, cp, w2, b2, w3, wc, b3, bc, out, accv):
    b = pl.program_id(0)
    nic = ni[...]
    w2full = w2[...]
    h = jnp.dot(p2a[...] * nic, w2full[:128], preferred_element_type=jnp.float32)
    h += jnp.dot(p2b[...] * nic, w2full[128:256], preferred_element_type=jnp.float32)
    h += jnp.dot(p2c[...] * nic, w2full[256:384], preferred_element_type=jnp.float32)
    h += jnp.dot(p2d[...] * nic, w2full[384:], preferred_element_type=jnp.float32)
    h = jnp.maximum(h + b2[...], 0.0)
    c = cp[0] + cp[1]
    w = (c * no[...]) * (1.0 / N)
    v = jnp.sum(h * w, axis=0, keepdims=True)

    @pl.when(b == 0)
    def _():
        accv[...] = jnp.zeros_like(accv)
    accv[...] += v

    @pl.when(b == 19)
    def _():
        hg = jnp.dot(accv[...], w3[...], preferred_element_type=jnp.float32) + b3[...]
        out[...] = jnp.dot(hg, wc[...], preferred_element_type=jnp.float32) + bc[...]


def _mlp2(p2a, p2b, p2c, p2d, ni2, no2, cp3, W2, b2, W3, Wc, b3, bc):
    return pl.pallas_call(
        _mlp2_body,
        grid=(20,),
        in_specs=[
            pl.BlockSpec((512, 128), lambda b: (b, 0)),
            pl.BlockSpec((512, 128), lambda b: (b, 0)),
            pl.BlockSpec((512, 128), lambda b: (b, 0)),
            pl.BlockSpec((512, 128), lambda b: (b, 0)),
            pl.BlockSpec((512, 1), lambda b: (b, 0)),
            pl.BlockSpec((512, 1), lambda b: (b, 0)),
            pl.BlockSpec((2, 512, 1), lambda b: (0, b, 0)),
            pl.BlockSpec((H, H), lambda b: (0, 0)),
            pl.BlockSpec((1, H), lambda b: (0, 0)),
            pl.BlockSpec((H, H), lambda b: (0, 0)),
            pl.BlockSpec((H, NCLS), lambda b: (0, 0)),
            pl.BlockSpec((1, H), lambda b: (0, 0)),
            pl.BlockSpec((1, NCLS), lambda b: (0, 0)),
        ],
        out_specs=pl.BlockSpec((1, NCLS), lambda b: (0, 0)),
        out_shape=jax.ShapeDtypeStruct((1, NCLS), jnp.float32),
        scratch_shapes=[pltpu.VMEM((1, H), jnp.float32)],
    )(p2a, p2b, p2c, p2d, ni2, no2, cp3, W2, b2, W3, Wc, b3, bc)


def kernel(in_feat, edge_index, W1, b1, W2, b2, W3, b3, Wc, bc):
    f32 = jnp.float32
    xpad = jnp.pad(in_feat, ((0, NP - N), (0, 0)))
    zrow = jnp.zeros((640,), f32)
    zstripe = jnp.zeros((648, 128), f32)
    esrc = edge_index[0]
    edst = edge_index[1]

    do_p, di_p = _deg_kernel(esrc, edst, zrow)
    do3 = do_p.reshape(2, NP, 1)
    di3 = di_p.reshape(2, NP, 1)
    x0a, x0b, no2, ni2 = _scale(xpad, do3, di3)
    (c_pg,) = _cvec_kernel(esrc, edst, ni2.reshape(NP), zrow)
    c_p = c_pg[:, :NP]
    p1a, p1b = _prop_kernel(x0a, x0b, esrc, edst, zstripe)
    x1a, x1b, x1c, x1d = _mlp1(p1a, p1b, ni2, no2, W1, b1.reshape(1, H))
    p2a, p2b = _prop_kernel(x1a, x1b, esrc, edst, zstripe)
    p2c, p2d = _prop_kernel(x1c, x1d, esrc, edst, zstripe)
    out = _mlp2(p2a, p2b, p2c, p2d, ni2, no2, c_p.reshape(2, NP, 1),
                W2, b2.reshape(1, H), W3, Wc, b3.reshape(1, H),
                bc.reshape(1, NCLS))
    return out


# speedup vs baseline: 8.2659x; 1.1333x over previous
import functools

import jax
import jax.numpy as jnp
from jax import lax
from jax.experimental import pallas as pl
from jax.experimental.pallas import tpu as pltpu
from jax.experimental.pallas import tpu_sc as plsc

N = 10000
E = 160000
F = 256
H = 512
NP = 10240
HALF = NP // 2
ACC_ROWS = HALF + 128
NCLS = 10

_mesh = plsc.VectorSubcoreMesh(core_axis_name="c", subcore_axis_name="s")


def _iota16():
    return lax.iota(jnp.int32, 16)


@functools.partial(
    pl.kernel, mesh=_mesh,
    out_type=[jax.ShapeDtypeStruct((2, NP), jnp.float32),
              jax.ShapeDtypeStruct((2, NP), jnp.float32)],
    scratch_types=[
        pltpu.VMEM((5008,), jnp.int32),
        pltpu.VMEM((5008,), jnp.int32),
        pltpu.VMEM((128,), jnp.int32),
        pltpu.VMEM((128,), jnp.int32),
        pltpu.VMEM((128,), jnp.float32),
        pltpu.VMEM((16,), jnp.int32),
        pltpu.VMEM((16,), jnp.int32),
        pltpu.VMEM((16,), jnp.float32),
        pltpu.VMEM_SHARED((NP,), jnp.float32),
        pltpu.VMEM_SHARED((NP,), jnp.float32),
    ],
)
def _deg_kernel(esrc, edst, zrow, do_out, di_out,
                sbuf, dbuf, iob, iib, onesb, tio, tii, tv, acco, acci):
    cid = lax.axis_index("c")
    sid = lax.axis_index("s")
    tid = cid * 16 + sid
    pltpu.sync_copy(zrow, acco.at[pl.ds(sid * 640, 640)])
    pltpu.sync_copy(zrow, acci.at[pl.ds(sid * 640, 640)])
    onev = jnp.ones((16,), jnp.float32)

    def fill_ones(i, _):
        onesb[pl.ds(i * 16, 16)] = onev
        return 0
    lax.fori_loop(0, 8, fill_ones, 0)
    z16 = jnp.zeros((16,), jnp.int32)
    sbuf[pl.ds(4992, 16)] = z16
    dbuf[pl.ds(4992, 16)] = z16
    base = tid * 5000
    pltpu.sync_copy(esrc.at[pl.ds(base, 5000)], sbuf.at[pl.ds(0, 5000)])
    pltpu.sync_copy(edst.at[pl.ds(base, 5000)], dbuf.at[pl.ds(0, 5000)])
    plsc.subcore_barrier()

    def batch(b, _):
        for j in range(8):
            off = b * 128 + 16 * j
            s16 = sbuf[pl.ds(off, 16)]
            d16 = dbuf[pl.ds(off, 16)]
            iob[pl.ds(16 * j, 16)] = jnp.clip(s16, 0, NP - 1)
            iib[pl.ds(16 * j, 16)] = jnp.clip(d16, 0, NP - 1)
        pltpu.sync_copy(onesb, acco.at[iob], add=True)
        pltpu.sync_copy(onesb, acci.at[iib], add=True)
        return 0
    lax.fori_loop(0, 39, batch, 0)
    valid = _iota16() < 8
    s16 = sbuf[pl.ds(4992, 16)]
    d16 = dbuf[pl.ds(4992, 16)]
    tio[...] = jnp.clip(s16, 0, NP - 1)
    tii[...] = jnp.clip(d16, 0, NP - 1)
    tv[...] = jnp.where(valid, 1.0, 0.0).astype(jnp.float32)
    pltpu.sync_copy(tv, acco.at[tio], add=True)
    pltpu.sync_copy(tv, acci.at[tii], add=True)
    plsc.subcore_barrier()
    pltpu.sync_copy(acco.at[pl.ds(sid * 640, 640)],
                    do_out.at[cid, pl.ds(sid * 640, 640)])
    pltpu.sync_copy(acci.at[pl.ds(sid * 640, 640)],
                    di_out.at[cid, pl.ds(sid * 640, 640)])


NPG = NP + 128


@functools.partial(
    pl.kernel, mesh=_mesh,
    out_type=[jax.ShapeDtypeStruct((2, NPG), jnp.float32)],
    scratch_types=[
        pltpu.VMEM((5120,), jnp.int32),
        pltpu.VMEM((5120,), jnp.int32),
        pltpu.VMEM((128,), jnp.int32),
        pltpu.VMEM((128,), jnp.int32),
        pltpu.VMEM((128,), jnp.int32),
        pltpu.VMEM((128,), jnp.int32),
        pltpu.VMEM((128,), jnp.float32),
        pltpu.VMEM((128,), jnp.float32),
        pltpu.VMEM_SHARED((NPG,), jnp.float32),
        pltpu.SemaphoreType.DMA,
        pltpu.SemaphoreType.DMA,
    ],
)
def _cvec_kernel(esrc, edst, ni, zrow, c_out,
                 sbuf, dbuf, ib0, ib1, db0, db1, vb0, vb1, acc, sem0, sem1):
    cid = lax.axis_index("c")
    sid = lax.axis_index("s")
    tid = cid * 16 + sid
    pltpu.sync_copy(zrow, acc.at[pl.ds(sid * 640, 640)])

    @pl.when(sid == 0)
    def _():
        pltpu.sync_copy(zrow.at[pl.ds(0, 128)], acc.at[pl.ds(NP, 128)])
    iot = _iota16()
    garb = NP + sid * 4 + (iot & 3)
    z16 = jnp.zeros((16,), jnp.int32)

    def padtail(k, _):
        sbuf[pl.ds(5008 + k * 16, 16)] = garb
        dbuf[pl.ds(5008 + k * 16, 16)] = z16
        return 0
    base = tid * 5000
    pltpu.sync_copy(esrc.at[pl.ds(base, 5000)], sbuf.at[pl.ds(0, 5000)])
    pltpu.sync_copy(edst.at[pl.ds(base, 5000)], dbuf.at[pl.ds(0, 5000)])
    sreal = sbuf[pl.ds(4992, 16)]
    dreal = dbuf[pl.ds(4992, 16)]
    mreal = iot < 8
    sbuf[pl.ds(4992, 16)] = jnp.where(mreal, sreal, garb)
    dbuf[pl.ds(4992, 16)] = jnp.where(mreal, dreal, z16)
    lax.fori_loop(0, 7, padtail, 0)
    plsc.subcore_barrier()
    ibs = (ib0, ib1)
    dbs = (db0, db1)
    vbs = (vb0, vb1)
    sems = (sem0, sem1)

    def build(b, slot):
        for j in range(8):
            off = b * 128 + 16 * j
            s16 = jnp.clip(sbuf[pl.ds(off, 16)], 0, NPG - 1)
            d16 = jnp.clip(dbuf[pl.ds(off, 16)], 0, NP - 1)
            ibs[slot][pl.ds(16 * j, 16)] = s16
            dbs[slot][pl.ds(16 * j, 16)] = d16

    def pair(p, _):
        build(2 * p, 0)
        g0 = pltpu.async_copy(ni.at[db0], vb0, sem0)
        build(2 * p + 1, 1)
        g1 = pltpu.async_copy(ni.at[db1], vb1, sem1)
        g0.wait()
        pltpu.sync_copy(vb0, acc.at[ib0], add=True)
        g1.wait()
        pltpu.sync_copy(vb1, acc.at[ib1], add=True)
        return 0
    lax.fori_loop(0, 20, pair, 0)
    plsc.subcore_barrier()
    pltpu.sync_copy(acc.at[pl.ds(sid * 640, 640)],
                    c_out.at[cid, pl.ds(sid * 640, 640)])

    @pl.when(sid == 0)
    def _():
        pltpu.sync_copy(acc.at[pl.ds(NP, 128)], c_out.at[cid, pl.ds(NP, 128)])


EPT = E // 16
NPR = NP + 128
NSLOT = 4
BEDGE = 64


@functools.partial(
    pl.kernel, mesh=_mesh,
    out_type=[jax.ShapeDtypeStruct((NP, 128), jnp.float32),
              jax.ShapeDtypeStruct((NP, 128), jnp.float32)],
    scratch_types=(
        [pltpu.VMEM((2048,), jnp.int32)] * 2
        + [pltpu.VMEM((BEDGE,), jnp.int32)] * (2 * NSLOT)
        + [pltpu.VMEM((BEDGE, 128), jnp.float32)] * NSLOT
        + [pltpu.VMEM_SHARED((NPR, 128), jnp.float32)]
        + [pltpu.SemaphoreType.DMA] * (2 * NSLOT)
    ),
)
def _prop_kernel(xa, xb, esrc, edst, zstripe, ya, yb,
                 sbuf, dbuf, gf0, gf1, gf2, gf3, df0, df1, df2, df3,
                 sa0, sa1, sa2, sa3, acc,
                 ma0, ma1, ma2, ma3, wa0, wa1, wa2, wa3):
    cid = lax.axis_index("c")
    sid = lax.axis_index("s")
    pltpu.sync_copy(zstripe, acc.at[pl.ds(sid * 648, 648)])
    plsc.subcore_barrier()
    iot = _iota16()
    garb = NP + sid * 8 + (iot & 7)
    gfs = (gf0, gf1, gf2, gf3)
    dfs = (df0, df1, df2, df3)
    sas = (sa0, sa1, sa2, sa3)
    mas = (ma0, ma1, ma2, ma3)
    was = (wa0, wa1, wa2, wa3)

    def build(b, slot):
        for j in range(BEDGE // 16):
            off = b * BEDGE + 16 * j
            s16 = jnp.clip(sbuf[pl.ds(off, 16)], 0, NP - 1)
            d16 = jnp.clip(dbuf[pl.ds(off, 16)], 0, NPR - 1)
            gfs[slot][pl.ds(16 * j, 16)] = s16
            dfs[slot][pl.ds(16 * j, 16)] = d16

    def scan(xref):
        def quad(q, _):
            cps = []
            for sslot in range(NSLOT):
                @pl.when(q > 0)
                def _(sslot=sslot):
                    pltpu.make_async_copy(
                        sas[sslot], acc.at[dfs[sslot]], was[sslot]).wait()
                build(NSLOT * q + sslot, sslot)
                cps.append(pltpu.async_copy(xref.at[gfs[sslot]],
                                            sas[sslot], mas[sslot]))
            for sslot in range(NSLOT):
                cps[sslot].wait()
                pltpu.async_copy(sas[sslot], acc.at[dfs[sslot]],
                                 was[sslot], add=True)
            return 0

        def drain():
            for sslot in range(NSLOT):
                pltpu.make_async_copy(
                    sas[sslot], acc.at[dfs[sslot]], was[sslot]).wait()

        def chunk(c, _):
            pltpu.sync_copy(esrc.at[pl.ds(sid * EPT + 2048 * c, 2048)], sbuf)
            pltpu.sync_copy(edst.at[pl.ds(sid * EPT + 2048 * c, 2048)], dbuf)
            lax.fori_loop(0, 2048 // (NSLOT * BEDGE), quad, 0)
            drain()
            return 0
        lax.fori_loop(0, 4, chunk, 0)

        def padtail(k, _):
            sbuf[pl.ds(1808 + k * 16, 16)] = iot
            dbuf[pl.ds(1808 + k * 16, 16)] = garb
            return 0
        lax.fori_loop(0, 15, padtail, 0)
        pltpu.sync_copy(esrc.at[pl.ds(sid * EPT + 8192, 1808)],
                        sbuf.at[pl.ds(0, 1808)])
        pltpu.sync_copy(edst.at[pl.ds(sid * EPT + 8192, 1808)],
                        dbuf.at[pl.ds(0, 1808)])
        lax.fori_loop(0, 2048 // (NSLOT * BEDGE), quad, 0)
        drain()

    @pl.when(cid == 0)
    def _():
        scan(xa)

    @pl.when(cid == 1)
    def _():
        scan(xb)
    plsc.subcore_barrier()
    row0 = sid * 640

    @pl.when(cid == 0)
    def _():
        for k in range(2):
            pltpu.sync_copy(acc.at[pl.ds(row0 + 320 * k, 320)],
                            ya.at[pl.ds(row0 + 320 * k, 320)])

    @pl.when(cid == 1)
    def _():
        for k in range(2):
            pltpu.sync_copy(acc.at[pl.ds(row0 + 320 * k, 320)],
                            yb.at[pl.ds(row0 + 320 * k, 320)])


def _scale_body(xf, dop, dip, x0a, x0b, no_o, ni_o):
    do = dop[0] + dop[1]
    di = dip[0] + dip[1]
    no = lax.rsqrt(jnp.clip(do, 1.0, None))
    ni = lax.rsqrt(jnp.clip(di, 1.0, None))
    no_o[...] = no
    ni_o[...] = ni
    x0 = xf[...] * no
    x0a[...] = x0[:, :128]
    x0b[...] = x0[:, 128:]


def _scale(xpad, do3, di3):
    return pl.pallas_call(
        _scale_body,
        grid=(16,),
        in_specs=[
            pl.BlockSpec((640, F), lambda b: (b, 0)),
            pl.BlockSpec((2, 640, 1), lambda b: (0, b, 0)),
            pl.BlockSpec((2, 640, 1), lambda b: (0, b, 0)),
        ],
        out_specs=[
            pl.BlockSpec((640, 128), lambda b: (b, 0)),
            pl.BlockSpec((640, 128), lambda b: (b, 0)),
            pl.BlockSpec((640, 1), lambda b: (b, 0)),
            pl.BlockSpec((640, 1), lambda b: (b, 0)),
        ],
        out_shape=[
            jax.ShapeDtypeStruct((NP, 128), jnp.float32),
            jax.ShapeDtypeStruct((NP, 128), jnp.float32),
            jax.ShapeDtypeStruct((NP, 1), jnp.float32),
            jax.ShapeDtypeStruct((NP, 1), jnp.float32),
        ],
    )(xpad, do3, di3)


def _mlp1_body(p1a, p1b, ni, no, w1, b1, xa, xb, xc, xd):
    nic = ni[...]
    noc = no[...]
    w1full = w1[...]
    h = jnp.dot(p1a[...] * nic, w1full[:128], preferred_element_type=jnp.float32)
    h += jnp.dot(p1b[...] * nic, w1full[128:], preferred_element_type=jnp.float32)
    h = jnp.maximum(h + b1[...], 0.0) * noc
    xa[...] = h[:, :128]
    xb[...] = h[:, 128:256]
    xc[...] = h[:, 256:384]
    xd[...] = h[:, 384:]


def _mlp1(p1a, p1b, ni2, no2, W1, b1):
    return pl.pallas_call(
        _mlp1_body,
        grid=(20,),
        in_specs=[
            pl.BlockSpec((512, 128), lambda b: (b, 0)),
            pl.BlockSpec((512, 128), lambda b: (b, 0)),
            pl.BlockSpec((512, 1), lambda b: (b, 0)),
            pl.BlockSpec((512, 1), lambda b: (b, 0)),
            pl.BlockSpec((F, H), lambda b: (0, 0)),
            pl.BlockSpec((1, H), lambda b: (0, 0)),
        ],
        out_specs=[
            pl.BlockSpec((512, 128), lambda b: (b, 0)),
            pl.BlockSpec((512, 128), lambda b: (b, 0)),
            pl.BlockSpec((512, 128), lambda b: (b, 0)),
            pl.BlockSpec((512, 128), lambda b: (b, 0)),
        ],
        out_shape=[
            jax.ShapeDtypeStruct((NP, 128), jnp.float32),
            jax.ShapeDtypeStruct((NP, 128), jnp.float32),
            jax.ShapeDtypeStruct((NP, 128), jnp.float32),
            jax.ShapeDtypeStruct((NP, 128), jnp.float32),
        ],
    )(p1a, p1b, ni2, no2, W1, b1)


def _mlp2_body(p2a, p2b, p2c, p2d, ni, no, cp, w2, b2, w3, wc, b3, bc, out, accv):
    b = pl.program_id(0)
    nic = ni[...]
    w2full = w2[...]
    h = jnp.dot(p2a[...] * nic, w2full[:128], preferred_element_type=jnp.float32)
    h += jnp.dot(p2b[...] * nic, w2full[128:256], preferred_element_type=jnp.float32)
    h += jnp.dot(p2c[...] * nic, w2full[256:384], preferred_element_type=jnp.float32)
    h += jnp.dot(p2d[...] * nic, w2full[384:], preferred_element_type=jnp.float32)
    h = jnp.maximum(h + b2[...], 0.0)
    c = cp[0] + cp[1]
    w = (c * no[...]) * (1.0 / N)
    v = jnp.sum(h * w, axis=0, keepdims=True)

    @pl.when(b == 0)
    def _():
        accv[...] = jnp.zeros_like(accv)
    accv[...] += v

    @pl.when(b == 19)
    def _():
        hg = jnp.dot(accv[...], w3[...], preferred_element_type=jnp.float32) + b3[...]
        out[...] = jnp.dot(hg, wc[...], preferred_element_type=jnp.float32) + bc[...]


def _mlp2(p2a, p2b, p2c, p2d, ni2, no2, cp3, W2, b2, W3, Wc, b3, bc):
    return pl.pallas_call(
        _mlp2_body,
        grid=(20,),
        in_specs=[
            pl.BlockSpec((512, 128), lambda b: (b, 0)),
            pl.BlockSpec((512, 128), lambda b: (b, 0)),
            pl.BlockSpec((512, 128), lambda b: (b, 0)),
            pl.BlockSpec((512, 128), lambda b: (b, 0)),
            pl.BlockSpec((512, 1), lambda b: (b, 0)),
            pl.BlockSpec((512, 1), lambda b: (b, 0)),
            pl.BlockSpec((2, 512, 1), lambda b: (0, b, 0)),
            pl.BlockSpec((H, H), lambda b: (0, 0)),
            pl.BlockSpec((1, H), lambda b: (0, 0)),
            pl.BlockSpec((H, H), lambda b: (0, 0)),
            pl.BlockSpec((H, NCLS), lambda b: (0, 0)),
            pl.BlockSpec((1, H), lambda b: (0, 0)),
            pl.BlockSpec((1, NCLS), lambda b: (0, 0)),
        ],
        out_specs=pl.BlockSpec((1, NCLS), lambda b: (0, 0)),
        out_shape=jax.ShapeDtypeStruct((1, NCLS), jnp.float32),
        scratch_shapes=[pltpu.VMEM((1, H), jnp.float32)],
    )(p2a, p2b, p2c, p2d, ni2, no2, cp3, W2, b2, W3, Wc, b3, bc)


def kernel(in_feat, edge_index, W1, b1, W2, b2, W3, b3, Wc, bc):
    f32 = jnp.float32
    xpad = jnp.pad(in_feat, ((0, NP - N), (0, 0)))
    zrow = jnp.zeros((640,), f32)
    zstripe = jnp.zeros((648, 128), f32)
    esrc = edge_index[0]
    edst = edge_index[1]

    do_p, di_p = _deg_kernel(esrc, edst, zrow)
    do3 = do_p.reshape(2, NP, 1)
    di3 = di_p.reshape(2, NP, 1)
    x0a, x0b, no2, ni2 = _scale(xpad, do3, di3)
    (c_pg,) = _cvec_kernel(esrc, edst, ni2.reshape(NP), zrow)
    c_p = c_pg[:, :NP]
    p1a, p1b = _prop_kernel(x0a, x0b, esrc, edst, zstripe)
    x1a, x1b, x1c, x1d = _mlp1(p1a, p1b, ni2, no2, W1, b1.reshape(1, H))
    p2a, p2b = _prop_kernel(x1a, x1b, esrc, edst, zstripe)
    p2c, p2d = _prop_kernel(x1c, x1d, esrc, edst, zstripe)
    out = _mlp2(p2a, p2b, p2c, p2d, ni2, no2, c_p.reshape(2, NP, 1),
                W2, b2.reshape(1, H), W3, Wc, b3.reshape(1, H),
                bc.reshape(1, NCLS))
    return out
